# Initial kernel scaffold; baseline (speedup 1.0000x reference)
import jax
import jax.numpy as jnp
from jax.experimental import pallas as pl


def kernel(delta_z, edge_index, edge_weight):
    raise NotImplementedError("write your pallas kernel here")


# speedup vs baseline: 8.5643x; 8.5643x over previous
import functools

import jax
import jax.numpy as jnp
from jax import lax
from jax.experimental import pallas as pl
from jax.experimental.pallas import tpu as pltpu
from jax.experimental.pallas import tpu_sc as plsc


def _pick_chunk(ew: int) -> int:
    for ck in range(min(ew, 12000), 15, -1):
        if ew % ck == 0 and ck % 16 == 0:
            return ck
    return ew


@functools.partial(jax.jit, static_argnames=("t", "n", "e"))
def _smoothness_sc(zt, row, col, w, *, t, n, e):
    info = plsc.get_sparse_core_info()
    nw = info.num_cores * info.num_subcores
    ew = e // nw
    ck = _pick_chunk(ew)
    nchunks = ew // ck
    ngroups = ck // 16

    mesh = plsc.VectorSubcoreMesh(core_axis_name="c", subcore_axis_name="s")

    @functools.partial(
        pl.kernel,
        mesh=mesh,
        compiler_params=pltpu.CompilerParams(needs_layout_passes=False),
        out_type=jax.ShapeDtypeStruct((nw, 16), jnp.float32),
        scratch_types=[
            pltpu.VMEM((n,), jnp.float32),
            pltpu.VMEM((ck,), jnp.int32),
            pltpu.VMEM((ck,), jnp.int32),
            pltpu.VMEM((ck,), jnp.float32),
            pltpu.VMEM((16,), jnp.float32),
        ],
    )
    def body(zt_hbm, row_hbm, col_hbm, w_hbm, out_hbm, ztab, rbuf, cbuf, wbuf, accv):
        cid = lax.axis_index("c")
        sid = lax.axis_index("s")
        wid = sid * info.num_cores + cid
        ebase = wid * ew

        acc = jnp.zeros((16,), jnp.float32)
        for tt in range(t):
            pltpu.sync_copy(zt_hbm.at[tt], ztab)

            def chunk_body(k, acc):
                base = ebase + k * ck
                pltpu.sync_copy(row_hbm.at[pl.ds(base, ck)], rbuf)
                pltpu.sync_copy(col_hbm.at[pl.ds(base, ck)], cbuf)
                pltpu.sync_copy(w_hbm.at[pl.ds(base, ck)], wbuf)

                def group_body(g, acc):
                    ri = rbuf[pl.ds(g * 16, 16)]
                    ci = cbuf[pl.ds(g * 16, 16)]
                    wv = wbuf[pl.ds(g * 16, 16)]
                    a = plsc.load_gather(ztab, [ri])
                    b = plsc.load_gather(ztab, [ci])
                    d = a - b
                    return acc + wv * d * d

                return lax.fori_loop(0, ngroups, group_body, acc)

            acc = lax.fori_loop(0, nchunks, chunk_body, acc)

        accv[...] = acc
        pltpu.sync_copy(accv, out_hbm.at[wid])

    return body(zt, row, col, w)


def kernel(delta_z, edge_index, edge_weight):
    t, n, _ = delta_z.shape
    e = edge_weight.shape[0]
    zt = delta_z.reshape(t, n)
    row = edge_index[0].astype(jnp.int32)
    col = edge_index[1].astype(jnp.int32)
    partials = _smoothness_sc(zt, row, col, edge_weight, t=t, n=n, e=e)
    return partials.sum() / jnp.float32(t * e)


# speedup vs baseline: 13.2582x; 1.5481x over previous
import functools

import jax
import jax.numpy as jnp
from jax import lax
from jax.experimental import pallas as pl
from jax.experimental.pallas import tpu as pltpu
from jax.experimental.pallas import tpu_sc as plsc


def _pick_chunk(ew: int) -> int:
    for ck in range(min(ew, 12000), 15, -1):
        if ew % ck == 0 and ck % 16 == 0:
            return ck
    return ew


@functools.partial(jax.jit, static_argnames=("tp", "n", "e"))
def _smoothness_sc(zpack, row, col, w, *, tp, n, e):
    info = plsc.get_sparse_core_info()
    nw = info.num_cores * info.num_subcores
    ew = e // nw
    ck = _pick_chunk(ew)
    nchunks = ew // ck
    ngroups = ck // 16

    mesh = plsc.VectorSubcoreMesh(core_axis_name="c", subcore_axis_name="s")

    @functools.partial(
        pl.kernel,
        mesh=mesh,
        compiler_params=pltpu.CompilerParams(needs_layout_passes=False),
        out_type=jax.ShapeDtypeStruct((nw, 16), jnp.float32),
        scratch_types=[
            pltpu.VMEM((n,), jnp.int32),
            pltpu.VMEM((ck,), jnp.int32),
            pltpu.VMEM((ck,), jnp.int32),
            pltpu.VMEM((ck,), jnp.float32),
            pltpu.VMEM((16,), jnp.float32),
        ],
    )
    def body(zp_hbm, row_hbm, col_hbm, w_hbm, out_hbm, ztab, rbuf, cbuf, wbuf, accv):
        cid = lax.axis_index("c")
        sid = lax.axis_index("s")
        wid = sid * info.num_cores + cid
        ebase = wid * ew

        acc = jnp.zeros((16,), jnp.float32)
        for p in range(tp):
            pltpu.sync_copy(zp_hbm.at[p], ztab)

            def chunk_body(k, acc):
                base = ebase + k * ck
                pltpu.sync_copy(row_hbm.at[pl.ds(base, ck)], rbuf)
                pltpu.sync_copy(col_hbm.at[pl.ds(base, ck)], cbuf)
                pltpu.sync_copy(w_hbm.at[pl.ds(base, ck)], wbuf)

                def group_body(g, acc):
                    ri = rbuf[pl.ds(g * 16, 16)]
                    ci = cbuf[pl.ds(g * 16, 16)]
                    wv = wbuf[pl.ds(g * 16, 16)]
                    aw = plsc.load_gather(ztab, [ri])
                    bw = plsc.load_gather(ztab, [ci])
                    a0, a1 = plsc.unpack(plsc.bitcast(aw, jnp.bfloat16),
                                         format=plsc.PackFormat.INTERLEAVED)
                    b0, b1 = plsc.unpack(plsc.bitcast(bw, jnp.bfloat16),
                                         format=plsc.PackFormat.INTERLEAVED)
                    d0 = a0 - b0
                    d1 = a1 - b1
                    return acc + wv * (d0 * d0 + d1 * d1)

                return lax.fori_loop(0, ngroups, group_body, acc)

            acc = lax.fori_loop(0, nchunks, chunk_body, acc)

        accv[...] = acc
        pltpu.sync_copy(accv, out_hbm.at[wid])

    return body(zpack, row, col, w)


def kernel(delta_z, edge_index, edge_weight):
    t, n, _ = delta_z.shape
    e = edge_weight.shape[0]
    zb = delta_z.reshape(t // 2, 2, n).transpose(0, 2, 1).astype(jnp.bfloat16)
    zpack = jax.lax.bitcast_convert_type(zb, jnp.int32)
    row = edge_index[0].astype(jnp.int32)
    col = edge_index[1].astype(jnp.int32)
    partials = _smoothness_sc(zpack, row, col, edge_weight, tp=t // 2, n=n, e=e)
    return partials.sum() / jnp.float32(t * e)


# speedup vs baseline: 15.3296x; 1.1562x over previous
import functools

import jax
import jax.numpy as jnp
from jax import lax
from jax.experimental import pallas as pl
from jax.experimental.pallas import tpu as pltpu
from jax.experimental.pallas import tpu_sc as plsc


def _pick_chunk(ew: int) -> int:
    for ck in range(min(ew, 12000), 15, -1):
        if ew % ck == 0 and ck % 16 == 0:
            return ck
    return ew


@functools.partial(jax.jit, static_argnames=("tp", "n", "e"))
def _smoothness_sc(zpack, row, col, w, *, tp, n, e):
    info = plsc.get_sparse_core_info()
    nw = info.num_cores * info.num_subcores
    ew = e // nw
    ck = _pick_chunk(ew)
    nchunks = ew // ck
    ngroups = ck // 16

    mesh = plsc.VectorSubcoreMesh(core_axis_name="c", subcore_axis_name="s")

    @functools.partial(
        pl.kernel,
        mesh=mesh,
        compiler_params=pltpu.CompilerParams(needs_layout_passes=False),
        out_type=jax.ShapeDtypeStruct((nw, 16), jnp.float32),
        scratch_types=[
            pltpu.VMEM((n,), jnp.int32),
            pltpu.VMEM((ck,), jnp.int32),
            pltpu.VMEM((ck,), jnp.int32),
            pltpu.VMEM((ck,), jnp.float32),
            pltpu.VMEM((16,), jnp.float32),
        ],
    )
    def body(zp_hbm, row_hbm, col_hbm, w_hbm, out_hbm, ztab, rbuf, cbuf, wbuf, accv):
        cid = lax.axis_index("c")
        sid = lax.axis_index("s")
        wid = sid * info.num_cores + cid
        ebase = wid * ew

        acc = jnp.zeros((16,), jnp.float32)
        for p in range(tp):
            pltpu.sync_copy(zp_hbm.at[p], ztab)

            def chunk_body(k, acc):
                base = ebase + k * ck
                pltpu.sync_copy(row_hbm.at[pl.ds(base, ck)], rbuf)
                pltpu.sync_copy(col_hbm.at[pl.ds(base, ck)], cbuf)
                pltpu.sync_copy(w_hbm.at[pl.ds(base, ck)], wbuf)

                @plsc.parallel_loop(0, ngroups, unroll=8, carry=acc)
                def group_loop(g, acc):
                    ri = rbuf[pl.ds(g * 16, 16)]
                    ci = cbuf[pl.ds(g * 16, 16)]
                    wv = wbuf[pl.ds(g * 16, 16)]
                    aw = plsc.load_gather(ztab, [ri])
                    bw = plsc.load_gather(ztab, [ci])
                    a0, a1 = plsc.unpack(plsc.bitcast(aw, jnp.bfloat16),
                                         format=plsc.PackFormat.INTERLEAVED)
                    b0, b1 = plsc.unpack(plsc.bitcast(bw, jnp.bfloat16),
                                         format=plsc.PackFormat.INTERLEAVED)
                    d0 = a0 - b0
                    d1 = a1 - b1
                    return acc + wv * (d0 * d0 + d1 * d1)

                return group_loop

            acc = lax.fori_loop(0, nchunks, chunk_body, acc)

        accv[...] = acc
        pltpu.sync_copy(accv, out_hbm.at[wid])

    return body(zpack, row, col, w)


def kernel(delta_z, edge_index, edge_weight):
    t, n, _ = delta_z.shape
    e = edge_weight.shape[0]
    zb = delta_z.reshape(t // 2, 2, n).transpose(0, 2, 1).astype(jnp.bfloat16)
    zpack = jax.lax.bitcast_convert_type(zb, jnp.int32)
    row = edge_index[0].astype(jnp.int32)
    col = edge_index[1].astype(jnp.int32)
    partials = _smoothness_sc(zpack, row, col, edge_weight, tp=t // 2, n=n, e=e)
    return partials.sum() / jnp.float32(t * e)


# speedup vs baseline: 21.4534x; 1.3995x over previous
import functools

import jax
import jax.numpy as jnp
from jax import lax
from jax.experimental import pallas as pl
from jax.experimental.pallas import tpu as pltpu
from jax.experimental.pallas import tpu_sc as plsc


def _pick_chunk(ew: int) -> int:
    for ck in range(min(ew, 12000), 15, -1):
        if ew % ck == 0 and ck % 16 == 0:
            return ck
    return ew


@functools.partial(jax.jit, static_argnames=("tp", "n", "e"))
def _smoothness_sc(zpack, row, col, w, *, tp, n, e):
    info = plsc.get_sparse_core_info()
    nw = info.num_cores * info.num_subcores
    ew = e // nw
    ck = _pick_chunk(ew)
    nchunks = ew // ck
    ngroups = ck // 16

    mesh = plsc.VectorSubcoreMesh(core_axis_name="c", subcore_axis_name="s")

    @functools.partial(
        pl.kernel,
        mesh=mesh,
        compiler_params=pltpu.CompilerParams(needs_layout_passes=False),
        out_type=jax.ShapeDtypeStruct((nw, 16), jnp.float32),
        scratch_types=[
            pltpu.VMEM((n,), jnp.int32),
            pltpu.VMEM((ck,), jnp.int32),
            pltpu.VMEM((ck,), jnp.int32),
            pltpu.VMEM((ck,), jnp.int32),
            pltpu.VMEM((ck,), jnp.int32),
            pltpu.VMEM((ck,), jnp.float32),
            pltpu.VMEM((ck,), jnp.float32),
            pltpu.VMEM((16,), jnp.float32),
            pltpu.SemaphoreType.DMA,
            pltpu.SemaphoreType.DMA,
        ],
    )
    def body(zp_hbm, row_hbm, col_hbm, w_hbm, out_hbm, ztab, rbuf0, rbuf1,
             cbuf0, cbuf1, wbuf0, wbuf1, accv, sem0, sem1):
        cid = lax.axis_index("c")
        sid = lax.axis_index("s")
        wid = sid * info.num_cores + cid
        ebase = wid * ew
        sems = (sem0, sem1)
        rbufs, cbufs, wbufs = (rbuf0, rbuf1), (cbuf0, cbuf1), (wbuf0, wbuf1)

        def fire(k, buf):
            base = ebase + k * ck
            sem = sems[buf]
            return (
                pltpu.async_copy(row_hbm.at[pl.ds(base, ck)], rbufs[buf], sem),
                pltpu.async_copy(col_hbm.at[pl.ds(base, ck)], cbufs[buf], sem),
                pltpu.async_copy(w_hbm.at[pl.ds(base, ck)], wbufs[buf], sem),
            )

        acc = jnp.zeros((16,), jnp.float32)

        def pass_body(p, acc):
            pltpu.sync_copy(zp_hbm.at[p], ztab)
            handles = fire(0, 0)
            for k in range(nchunks):
                cur = k % 2
                if k + 1 < nchunks:
                    next_handles = fire(k + 1, 1 - cur)
                for h in handles:
                    h.wait()
                if k + 1 < nchunks:
                    handles = next_handles

                @plsc.parallel_loop(0, ngroups, unroll=8, carry=acc)
                def group_loop(g, acc):
                    ri = rbufs[cur][pl.ds(g * 16, 16)]
                    ci = cbufs[cur][pl.ds(g * 16, 16)]
                    wv = wbufs[cur][pl.ds(g * 16, 16)]
                    aw = plsc.load_gather(ztab, [ri])
                    bw = plsc.load_gather(ztab, [ci])
                    a0, a1 = plsc.unpack(plsc.bitcast(aw, jnp.bfloat16),
                                         format=plsc.PackFormat.INTERLEAVED)
                    b0, b1 = plsc.unpack(plsc.bitcast(bw, jnp.bfloat16),
                                         format=plsc.PackFormat.INTERLEAVED)
                    d0 = a0 - b0
                    d1 = a1 - b1
                    return acc + wv * (d0 * d0 + d1 * d1)

                acc = group_loop
            return acc

        acc = lax.fori_loop(0, tp, pass_body, acc)

        accv[...] = acc
        pltpu.sync_copy(accv, out_hbm.at[wid])

    return body(zpack, row, col, w)


def kernel(delta_z, edge_index, edge_weight):
    t, n, _ = delta_z.shape
    e = edge_weight.shape[0]
    zb = delta_z.reshape(t // 2, 2, n).transpose(0, 2, 1).astype(jnp.bfloat16)
    zpack = jax.lax.bitcast_convert_type(zb, jnp.int32)
    row = edge_index[0].astype(jnp.int32)
    col = edge_index[1].astype(jnp.int32)
    partials = _smoothness_sc(zpack, row, col, edge_weight, tp=t // 2, n=n, e=e)
    return partials.sum() / jnp.float32(t * e)


# speedup vs baseline: 23.6505x; 1.1024x over previous
import functools

import jax
import jax.numpy as jnp
from jax import lax
from jax.experimental import pallas as pl
from jax.experimental.pallas import tpu as pltpu
from jax.experimental.pallas import tpu_sc as plsc


def _pick_chunk(ew: int, limit: int) -> int:
    for ck in range(min(ew, limit), 15, -1):
        if ew % ck == 0 and ck % 16 == 0:
            return ck
    return ew


@functools.partial(jax.jit, static_argnames=("tq", "n", "e"))
def _smoothness_sc(zpack, rc, w, *, tq, n, e):
    info = plsc.get_sparse_core_info()
    nw = info.num_cores * info.num_subcores
    ew = e // nw
    ck = _pick_chunk(ew, 4000)
    nchunks = ew // ck
    ngroups = ck // 16
    unroll = 8 if ngroups % 8 == 0 else (5 if ngroups % 5 == 0 else 1)

    mesh = plsc.VectorSubcoreMesh(core_axis_name="c", subcore_axis_name="s")

    @functools.partial(
        pl.kernel,
        mesh=mesh,
        compiler_params=pltpu.CompilerParams(needs_layout_passes=False),
        out_type=jax.ShapeDtypeStruct((nw, 16), jnp.float32),
        scratch_types=[
            pltpu.VMEM((n,), jnp.int32),
            pltpu.VMEM((n,), jnp.int32),
            pltpu.VMEM((ck,), jnp.int32),
            pltpu.VMEM((ck,), jnp.int32),
            pltpu.VMEM((ck,), jnp.float32),
            pltpu.VMEM((ck,), jnp.float32),
            pltpu.VMEM((16,), jnp.float32),
            pltpu.SemaphoreType.DMA,
            pltpu.SemaphoreType.DMA,
        ],
    )
    def body(zp_hbm, rc_hbm, w_hbm, out_hbm, ztab0, ztab1, rcb0, rcb1,
             wb0, wb1, accv, sem0, sem1):
        cid = lax.axis_index("c")
        sid = lax.axis_index("s")
        wid = sid * info.num_cores + cid
        ebase = wid * ew
        sems = (sem0, sem1)
        rcbufs, wbufs = (rcb0, rcb1), (wb0, wb1)

        def fire(k, buf):
            base = ebase + k * ck
            sem = sems[buf]
            return (
                pltpu.async_copy(rc_hbm.at[pl.ds(base, ck)], rcbufs[buf], sem),
                pltpu.async_copy(w_hbm.at[pl.ds(base, ck)], wbufs[buf], sem),
            )

        acc = jnp.zeros((16,), jnp.float32)

        def pass_body(q, acc):
            pltpu.sync_copy(zp_hbm.at[2 * q], ztab0)
            pltpu.sync_copy(zp_hbm.at[2 * q + 1], ztab1)
            handles = fire(0, 0)
            for k in range(nchunks):
                cur = k % 2
                if k + 1 < nchunks:
                    next_handles = fire(k + 1, 1 - cur)
                for h in handles:
                    h.wait()
                if k + 1 < nchunks:
                    handles = next_handles

                @plsc.parallel_loop(0, ngroups, unroll=unroll, carry=acc)
                def group_loop(g, acc):
                    rcv = rcbufs[cur][pl.ds(g * 16, 16)]
                    wv = wbufs[cur][pl.ds(g * 16, 16)]
                    ri = rcv & 0xFFFF
                    ci = lax.shift_right_logical(rcv, 16)
                    s = jnp.zeros((16,), jnp.float32)
                    for ztab in (ztab0, ztab1):
                        aw = plsc.load_gather(ztab, [ri])
                        bw = plsc.load_gather(ztab, [ci])
                        a0, a1 = plsc.unpack(plsc.bitcast(aw, jnp.bfloat16),
                                             format=plsc.PackFormat.INTERLEAVED)
                        b0, b1 = plsc.unpack(plsc.bitcast(bw, jnp.bfloat16),
                                             format=plsc.PackFormat.INTERLEAVED)
                        d0 = a0 - b0
                        d1 = a1 - b1
                        s = s + (d0 * d0 + d1 * d1)
                    return acc + wv * s

                acc = group_loop
            return acc

        acc = lax.fori_loop(0, tq, pass_body, acc)

        accv[...] = acc
        pltpu.sync_copy(accv, out_hbm.at[wid])

    return body(zpack, rc, w)


def kernel(delta_z, edge_index, edge_weight):
    t, n, _ = delta_z.shape
    e = edge_weight.shape[0]
    zb = delta_z.reshape(t // 2, 2, n).transpose(0, 2, 1).astype(jnp.bfloat16)
    zpack = jax.lax.bitcast_convert_type(zb, jnp.int32)
    row = edge_index[0].astype(jnp.uint32)
    col = edge_index[1].astype(jnp.uint32)
    rc = jax.lax.bitcast_convert_type(row | (col << 16), jnp.int32)
    partials = _smoothness_sc(zpack, rc, edge_weight, tq=t // 4, n=n, e=e)
    return partials.sum() / jnp.float32(t * e)
